# baseline (speedup 1.0000x reference)
import functools

import jax
import jax.numpy as jnp
from jax import lax
from jax.experimental import pallas as pl
from jax.experimental.pallas import tpu as pltpu
import jax.experimental.pallas.tpu_sc as plsc

_E = 8
_TOPK = 2
_N = 8192
_D = 1024
_DFF = 4096
_A = _N * _TOPK

_TM = 512
_TKD = 512
_NT = _A // _TM
_G = _NT + _E - 1
_KD = _DFF // _TKD

_NC = 2
_NS = 16
_NW = _NC * _NS


def _dest_body(ei_ref, cnt_ref, dest_ref):
    ei = ei_ref[...]

    r = lax.broadcasted_iota(jnp.int32, (128, 128), 0)
    c = lax.broadcasted_iota(jnp.int32, (128, 128), 1)
    tlt = (r < c).astype(jnp.float32)
    slo = (c < r).astype(jnp.float32)

    dest = jnp.zeros((128, 128), jnp.float32)
    off = jnp.int32(0)
    for e in range(_E):
        plane = (ei == e).astype(jnp.float32)
        col_ex = jnp.dot(plane, tlt, preferred_element_type=jnp.float32)
        rowsum = jnp.sum(plane, axis=1, keepdims=True)
        row_ex = jnp.dot(slo, rowsum, preferred_element_type=jnp.float32)
        rank = col_ex + row_ex
        dest = dest + plane * (off.astype(jnp.float32) + rank)
        off = off + cnt_ref[0, e]
    dest_ref[...] = dest.astype(jnp.int32)


def _dest(ei2d, cnt):
    return pl.pallas_call(
        _dest_body,
        in_specs=[
            pl.BlockSpec((128, 128), lambda: (0, 0)),
            pl.BlockSpec(memory_space=pltpu.SMEM),
        ],
        out_shape=jax.ShapeDtypeStruct((128, 128), jnp.int32),
    )(ei2d, cnt)


def _dispatch(x, dest01):
    tpw = _N // _NW
    ch = 64
    nch = tpw // ch
    mesh = plsc.VectorSubcoreMesh(core_axis_name="c", subcore_axis_name="s")

    @functools.partial(
        pl.kernel,
        out_type=jax.ShapeDtypeStruct((_A, _D), jnp.float32),
        mesh=mesh,
        scratch_types=[
            pltpu.VMEM((ch, _D), jnp.float32),
            pltpu.VMEM((ch,), jnp.int32),
            pltpu.VMEM((ch,), jnp.int32),
            pltpu.SemaphoreType.DMA,
            pltpu.SemaphoreType.DMA,
        ],
    )
    def k(x_hbm, d_hbm, xs_hbm, rows_v, i0_v, i1_v, s0, s1):
        wid = lax.axis_index("s") * _NC + lax.axis_index("c")
        base = wid * tpw
        for cnk in range(nch):
            t0 = base + cnk * ch
            pltpu.sync_copy(x_hbm.at[pl.ds(t0, ch)], rows_v)
            pltpu.sync_copy(d_hbm.at[0, pl.ds(t0, ch)], i0_v)
            pltpu.sync_copy(d_hbm.at[1, pl.ds(t0, ch)], i1_v)
            c0 = pltpu.async_copy(rows_v, xs_hbm.at[i0_v], s0)
            c1 = pltpu.async_copy(rows_v, xs_hbm.at[i1_v], s1)
            c0.wait()
            c1.wait()

    return k(x, dest01)


def _gmm_body(tile_ref, e_ref, lo_ref, hi_ref, xs_ref, w1_ref, w2_ref, ys_ref):
    g = pl.program_id(0)
    kd = pl.program_id(1)
    tile = tile_ref[g]
    rid = tile * _TM + lax.broadcasted_iota(jnp.int32, (_TM, 1), 0)
    m = (rid >= lo_ref[g]) & (rid < hi_ref[g])
    xblk = jnp.where(m, xs_ref[...], 0.0).astype(jnp.bfloat16)
    h = jnp.maximum(
        jnp.dot(xblk, w1_ref[0], preferred_element_type=jnp.float32), 0.0)
    y = jnp.dot(h.astype(jnp.bfloat16), w2_ref[0],
                preferred_element_type=jnp.float32)

    first = (kd == 0) & ((g == 0) | (tile != tile_ref[jnp.maximum(g - 1, 0)]))

    @pl.when(first)
    def _():
        ys_ref[...] = y

    @pl.when(jnp.logical_not(first))
    def _():
        ys_ref[...] += y


def _gmm(xs, w1, w2, tile_of, e_of, lo, hi):
    spec = pltpu.PrefetchScalarGridSpec(
        num_scalar_prefetch=4,
        grid=(_G, _KD),
        in_specs=[
            pl.BlockSpec((_TM, _D), lambda g, kd, t, e, lo, hi: (t[g], 0)),
            pl.BlockSpec((1, _D, _TKD), lambda g, kd, t, e, lo, hi: (e[g], 0, kd)),
            pl.BlockSpec((1, _TKD, _D), lambda g, kd, t, e, lo, hi: (e[g], kd, 0)),
        ],
        out_specs=pl.BlockSpec((_TM, _D), lambda g, kd, t, e, lo, hi: (t[g], 0)),
    )
    return pl.pallas_call(
        _gmm_body,
        grid_spec=spec,
        out_shape=jax.ShapeDtypeStruct((_A, _D), jnp.float32),
    )(tile_of, e_of, lo, hi, xs, w1, w2)


def _combine_gather(ys, dest01):
    tpw = _N // _NW
    ch = 32
    nch = tpw // ch
    mesh = plsc.VectorSubcoreMesh(core_axis_name="c", subcore_axis_name="s")
    sds = jax.ShapeDtypeStruct((_N, _D), jnp.float32)

    @functools.partial(
        pl.kernel,
        out_type=(sds, sds),
        mesh=mesh,
        scratch_types=[
            pltpu.VMEM((ch, _D), jnp.float32),
            pltpu.VMEM((ch, _D), jnp.float32),
            pltpu.VMEM((ch,), jnp.int32),
            pltpu.VMEM((ch,), jnp.int32),
            pltpu.SemaphoreType.DMA,
            pltpu.SemaphoreType.DMA,
        ],
    )
    def k(ys_hbm, d_hbm, y0_hbm, y1_hbm, r0_v, r1_v, i0_v, i1_v, s0, s1):
        wid = lax.axis_index("s") * _NC + lax.axis_index("c")
        base = wid * tpw
        for cnk in range(nch):
            t0 = base + cnk * ch
            pltpu.sync_copy(d_hbm.at[0, pl.ds(t0, ch)], i0_v)
            pltpu.sync_copy(d_hbm.at[1, pl.ds(t0, ch)], i1_v)
            c0 = pltpu.async_copy(ys_hbm.at[i0_v], r0_v, s0)
            c1 = pltpu.async_copy(ys_hbm.at[i1_v], r1_v, s1)
            c0.wait()
            c1.wait()
            pltpu.sync_copy(r0_v, y0_hbm.at[pl.ds(t0, ch)])
            pltpu.sync_copy(r1_v, y1_hbm.at[pl.ds(t0, ch)])

    return k(ys, dest01)


def _combine_body(y0_ref, y1_ref, ew_ref, out_ref):
    ew = ew_ref[...]
    out_ref[...] = y0_ref[...] * ew[:, 0:1] + y1_ref[...] * ew[:, 1:2]


def _combine(y0, y1, ew):
    tme = 512
    return pl.pallas_call(
        _combine_body,
        grid=(_N // tme,),
        in_specs=[
            pl.BlockSpec((tme, _D), lambda i: (i, 0)),
            pl.BlockSpec((tme, _D), lambda i: (i, 0)),
            pl.BlockSpec((tme, _TOPK), lambda i: (i, 0)),
        ],
        out_specs=pl.BlockSpec((tme, _D), lambda i: (i, 0)),
        out_shape=jax.ShapeDtypeStruct((_N, _D), jnp.float32),
    )(y0, y1, ew)


def _route_meta(counts):
    counts = counts.astype(jnp.int32)
    offs = jnp.concatenate([jnp.zeros((1,), jnp.int32), jnp.cumsum(counts)])
    start_t = offs[:_E] // _TM
    end_t = jnp.where(counts > 0, (offs[1:] - 1) // _TM, start_t - 1)
    nt = jnp.maximum(end_t - start_t + 1, 0)
    unit_start = jnp.concatenate([jnp.zeros((1,), jnp.int32), jnp.cumsum(nt)])
    total_units = unit_start[_E]
    g = jnp.arange(_G, dtype=jnp.int32)
    e_id = jnp.sum(
        (g[:, None] >= unit_start[None, 1:]).astype(jnp.int32), axis=1)
    valid = g < total_units
    e_id = jnp.clip(e_id, 0, _E - 1)
    tile_real = start_t[e_id] + (g - unit_start[e_id])
    tile_of = jnp.where(valid, tile_real, _NT - 1)
    lo = jnp.where(valid, jnp.maximum(offs[e_id], tile_of * _TM), 0)
    hi = jnp.where(valid, jnp.minimum(offs[e_id + 1], (tile_of + 1) * _TM), 0)
    e_of = jnp.where(valid, e_id, 0)
    return tile_of, e_of, lo, hi


def kernel(x, expert_weights, W1, W2, expert_indices, batch_size_per_expert):
    ei2d = expert_indices.reshape(-1).astype(jnp.int32).reshape(128, 128)
    cnt = batch_size_per_expert.reshape(1, _E).astype(jnp.int32)

    dest2d = _dest(ei2d, cnt)
    dest01 = dest2d.reshape(_N, _TOPK).T

    xs = _dispatch(x, dest01)
    meta = _route_meta(batch_size_per_expert)
    ys = _gmm(xs, W1.astype(jnp.bfloat16), W2.astype(jnp.bfloat16), *meta)
    yu0, yu1 = _combine_gather(ys, dest01)
    return _combine(yu0, yu1, expert_weights)


# speedup vs baseline: 1.2050x; 1.2050x over previous
import functools

import jax
import jax.numpy as jnp
from jax import lax
from jax.experimental import pallas as pl
from jax.experimental.pallas import tpu as pltpu
import jax.experimental.pallas.tpu_sc as plsc

_E = 8
_TOPK = 2
_N = 8192
_D = 1024
_DFF = 4096
_A = _N * _TOPK

_TM = 512
_TKD = 1024
_NT = _A // _TM
_G = _NT + _E - 1
_KD = _DFF // _TKD

_NC = 2
_NS = 16
_NW = _NC * _NS


def _dest_body(ei_ref, cnt_ref, dest_ref):
    ei = ei_ref[...]

    r = lax.broadcasted_iota(jnp.int32, (128, 128), 0)
    c = lax.broadcasted_iota(jnp.int32, (128, 128), 1)
    tlt = (r < c).astype(jnp.float32)
    slo = (c < r).astype(jnp.float32)

    dest = jnp.zeros((128, 128), jnp.float32)
    off = jnp.int32(0)
    for e in range(_E):
        plane = (ei == e).astype(jnp.float32)
        col_ex = jnp.dot(plane, tlt, preferred_element_type=jnp.float32)
        rowsum = jnp.sum(plane, axis=1, keepdims=True)
        row_ex = jnp.dot(slo, rowsum, preferred_element_type=jnp.float32)
        rank = col_ex + row_ex
        dest = dest + plane * (off.astype(jnp.float32) + rank)
        off = off + cnt_ref[0, e]
    dest_ref[...] = dest.astype(jnp.int32)


def _dest(ei2d, cnt):
    return pl.pallas_call(
        _dest_body,
        in_specs=[
            pl.BlockSpec((128, 128), lambda: (0, 0)),
            pl.BlockSpec(memory_space=pltpu.SMEM),
        ],
        out_shape=jax.ShapeDtypeStruct((128, 128), jnp.int32),
    )(ei2d, cnt)


def _dispatch(x, dest01):
    tpw = _N // _NW
    ch = 64
    nch = tpw // ch
    mesh = plsc.VectorSubcoreMesh(core_axis_name="c", subcore_axis_name="s")

    @functools.partial(
        pl.kernel,
        out_type=jax.ShapeDtypeStruct((_A, _D), jnp.float32),
        mesh=mesh,
        scratch_types=[
            pltpu.VMEM((ch, _D), jnp.float32),
            pltpu.VMEM((ch,), jnp.int32),
            pltpu.VMEM((ch,), jnp.int32),
            pltpu.SemaphoreType.DMA,
            pltpu.SemaphoreType.DMA,
        ],
    )
    def k(x_hbm, d_hbm, xs_hbm, rows_v, i0_v, i1_v, s0, s1):
        wid = lax.axis_index("s") * _NC + lax.axis_index("c")
        base = wid * tpw
        for cnk in range(nch):
            t0 = base + cnk * ch
            pltpu.sync_copy(x_hbm.at[pl.ds(t0, ch)], rows_v)
            pltpu.sync_copy(d_hbm.at[0, pl.ds(t0, ch)], i0_v)
            pltpu.sync_copy(d_hbm.at[1, pl.ds(t0, ch)], i1_v)
            c0 = pltpu.async_copy(rows_v, xs_hbm.at[i0_v], s0)
            c1 = pltpu.async_copy(rows_v, xs_hbm.at[i1_v], s1)
            c0.wait()
            c1.wait()

    return k(x, dest01)


def _gmm_body(tile_ref, e_ref, lo_ref, hi_ref, xs_ref, w1_ref, w2_ref, ys_ref):
    g = pl.program_id(0)
    kd = pl.program_id(1)
    tile = tile_ref[g]
    rid = tile * _TM + lax.broadcasted_iota(jnp.int32, (_TM, 1), 0)
    m = (rid >= lo_ref[g]) & (rid < hi_ref[g])
    xblk = jnp.where(m, xs_ref[...], 0.0)
    h = jnp.maximum(
        jnp.dot(xblk, w1_ref[0], preferred_element_type=jnp.float32), 0.0)
    y = jnp.dot(h, w2_ref[0], preferred_element_type=jnp.float32)

    first = (kd == 0) & ((g == 0) | (tile != tile_ref[jnp.maximum(g - 1, 0)]))

    @pl.when(first)
    def _():
        ys_ref[...] = y

    @pl.when(jnp.logical_not(first))
    def _():
        ys_ref[...] += y


def _gmm(xs, w1, w2, tile_of, e_of, lo, hi):
    spec = pltpu.PrefetchScalarGridSpec(
        num_scalar_prefetch=4,
        grid=(_G, _KD),
        in_specs=[
            pl.BlockSpec((_TM, _D), lambda g, kd, t, e, lo, hi: (t[g], 0)),
            pl.BlockSpec((1, _D, _TKD), lambda g, kd, t, e, lo, hi: (e[g], 0, kd)),
            pl.BlockSpec((1, _TKD, _D), lambda g, kd, t, e, lo, hi: (e[g], kd, 0)),
        ],
        out_specs=pl.BlockSpec((_TM, _D), lambda g, kd, t, e, lo, hi: (t[g], 0)),
    )
    return pl.pallas_call(
        _gmm_body,
        grid_spec=spec,
        out_shape=jax.ShapeDtypeStruct((_A, _D), jnp.float32),
    )(tile_of, e_of, lo, hi, xs, w1, w2)


def _combine_gather(ys, dest01):
    tpw = _N // _NW
    ch = 32
    nch = tpw // ch
    mesh = plsc.VectorSubcoreMesh(core_axis_name="c", subcore_axis_name="s")
    sds = jax.ShapeDtypeStruct((_N, _D), jnp.float32)

    @functools.partial(
        pl.kernel,
        out_type=(sds, sds),
        mesh=mesh,
        scratch_types=[
            pltpu.VMEM((ch, _D), jnp.float32),
            pltpu.VMEM((ch, _D), jnp.float32),
            pltpu.VMEM((ch,), jnp.int32),
            pltpu.VMEM((ch,), jnp.int32),
            pltpu.SemaphoreType.DMA,
            pltpu.SemaphoreType.DMA,
        ],
    )
    def k(ys_hbm, d_hbm, y0_hbm, y1_hbm, r0_v, r1_v, i0_v, i1_v, s0, s1):
        wid = lax.axis_index("s") * _NC + lax.axis_index("c")
        base = wid * tpw
        for cnk in range(nch):
            t0 = base + cnk * ch
            pltpu.sync_copy(d_hbm.at[0, pl.ds(t0, ch)], i0_v)
            pltpu.sync_copy(d_hbm.at[1, pl.ds(t0, ch)], i1_v)
            c0 = pltpu.async_copy(ys_hbm.at[i0_v], r0_v, s0)
            c1 = pltpu.async_copy(ys_hbm.at[i1_v], r1_v, s1)
            c0.wait()
            c1.wait()
            pltpu.sync_copy(r0_v, y0_hbm.at[pl.ds(t0, ch)])
            pltpu.sync_copy(r1_v, y1_hbm.at[pl.ds(t0, ch)])

    return k(ys, dest01)


def _combine_body(y0_ref, y1_ref, ew_ref, out_ref):
    ew = ew_ref[...]
    out_ref[...] = y0_ref[...] * ew[:, 0:1] + y1_ref[...] * ew[:, 1:2]


def _combine(y0, y1, ew):
    tme = 512
    return pl.pallas_call(
        _combine_body,
        grid=(_N // tme,),
        in_specs=[
            pl.BlockSpec((tme, _D), lambda i: (i, 0)),
            pl.BlockSpec((tme, _D), lambda i: (i, 0)),
            pl.BlockSpec((tme, _TOPK), lambda i: (i, 0)),
        ],
        out_specs=pl.BlockSpec((tme, _D), lambda i: (i, 0)),
        out_shape=jax.ShapeDtypeStruct((_N, _D), jnp.float32),
    )(y0, y1, ew)


def _route_meta(counts):
    counts = counts.astype(jnp.int32)
    offs = jnp.concatenate([jnp.zeros((1,), jnp.int32), jnp.cumsum(counts)])
    start_t = offs[:_E] // _TM
    end_t = jnp.where(counts > 0, (offs[1:] - 1) // _TM, start_t - 1)
    nt = jnp.maximum(end_t - start_t + 1, 0)
    unit_start = jnp.concatenate([jnp.zeros((1,), jnp.int32), jnp.cumsum(nt)])
    total_units = unit_start[_E]
    g = jnp.arange(_G, dtype=jnp.int32)
    e_id = jnp.sum(
        (g[:, None] >= unit_start[None, 1:]).astype(jnp.int32), axis=1)
    valid = g < total_units
    e_id = jnp.clip(e_id, 0, _E - 1)
    tile_real = start_t[e_id] + (g - unit_start[e_id])
    tile_of = jnp.where(valid, tile_real, _NT - 1)
    lo = jnp.where(valid, jnp.maximum(offs[e_id], tile_of * _TM), 0)
    hi = jnp.where(valid, jnp.minimum(offs[e_id + 1], (tile_of + 1) * _TM), 0)
    e_of = jnp.where(valid, e_id, 0)
    return tile_of, e_of, lo, hi


def kernel(x, expert_weights, W1, W2, expert_indices, batch_size_per_expert):
    ei2d = expert_indices.reshape(-1).astype(jnp.int32).reshape(128, 128)
    cnt = batch_size_per_expert.reshape(1, _E).astype(jnp.int32)

    dest2d = _dest(ei2d, cnt)
    dest01 = dest2d.reshape(_N, _TOPK).T

    xs = _dispatch(x, dest01)
    meta = _route_meta(batch_size_per_expert)
    ys = _gmm(xs, W1, W2, *meta)
    yu0, yu1 = _combine_gather(ys, dest01)
    return _combine(yu0, yu1, expert_weights)


# speedup vs baseline: 1.2771x; 1.0598x over previous
import functools

import jax
import jax.numpy as jnp
from jax import lax
from jax.experimental import pallas as pl
from jax.experimental.pallas import tpu as pltpu
import jax.experimental.pallas.tpu_sc as plsc

_E = 8
_TOPK = 2
_N = 8192
_D = 1024
_DFF = 4096
_A = _N * _TOPK

_TM = 1024
_TKD = 1024
_NT = _A // _TM
_G = _NT + _E - 1
_KD = _DFF // _TKD

_NC = 2
_NS = 16
_NW = _NC * _NS


def _dest_body(ei_ref, cnt_ref, dest_ref):
    ei = ei_ref[...]

    r = lax.broadcasted_iota(jnp.int32, (128, 128), 0)
    c = lax.broadcasted_iota(jnp.int32, (128, 128), 1)
    tlt = (r < c).astype(jnp.float32)
    slo = (c < r).astype(jnp.float32)

    dest = jnp.zeros((128, 128), jnp.float32)
    off = jnp.int32(0)
    for e in range(_E):
        plane = (ei == e).astype(jnp.float32)
        col_ex = jnp.dot(plane, tlt, preferred_element_type=jnp.float32)
        rowsum = jnp.sum(plane, axis=1, keepdims=True)
        row_ex = jnp.dot(slo, rowsum, preferred_element_type=jnp.float32)
        rank = col_ex + row_ex
        dest = dest + plane * (off.astype(jnp.float32) + rank)
        off = off + cnt_ref[0, e]
    dest_ref[...] = dest.astype(jnp.int32)


def _dest(ei2d, cnt):
    return pl.pallas_call(
        _dest_body,
        in_specs=[
            pl.BlockSpec((128, 128), lambda: (0, 0)),
            pl.BlockSpec(memory_space=pltpu.SMEM),
        ],
        out_shape=jax.ShapeDtypeStruct((128, 128), jnp.int32),
    )(ei2d, cnt)


def _dispatch(x, dest01):
    tpw = _N // _NW
    ch = 64
    nch = tpw // ch
    mesh = plsc.VectorSubcoreMesh(core_axis_name="c", subcore_axis_name="s")

    @functools.partial(
        pl.kernel,
        out_type=jax.ShapeDtypeStruct((_A, _D), jnp.float32),
        mesh=mesh,
        scratch_types=[
            pltpu.VMEM((ch, _D), jnp.float32),
            pltpu.VMEM((ch,), jnp.int32),
            pltpu.VMEM((ch,), jnp.int32),
            pltpu.SemaphoreType.DMA,
            pltpu.SemaphoreType.DMA,
        ],
    )
    def k(x_hbm, d_hbm, xs_hbm, rows_v, i0_v, i1_v, s0, s1):
        wid = lax.axis_index("s") * _NC + lax.axis_index("c")
        base = wid * tpw
        for cnk in range(nch):
            t0 = base + cnk * ch
            pltpu.sync_copy(x_hbm.at[pl.ds(t0, ch)], rows_v)
            pltpu.sync_copy(d_hbm.at[0, pl.ds(t0, ch)], i0_v)
            pltpu.sync_copy(d_hbm.at[1, pl.ds(t0, ch)], i1_v)
            c0 = pltpu.async_copy(rows_v, xs_hbm.at[i0_v], s0)
            c1 = pltpu.async_copy(rows_v, xs_hbm.at[i1_v], s1)
            c0.wait()
            c1.wait()

    return k(x, dest01)


def _gmm_body(tile_ref, e_ref, lo_ref, hi_ref, xs_ref, w1_ref, w2_ref, ys_ref):
    g = pl.program_id(0)
    kd = pl.program_id(1)
    tile = tile_ref[g]
    rid = tile * _TM + lax.broadcasted_iota(jnp.int32, (_TM, 1), 0)
    m = (rid >= lo_ref[g]) & (rid < hi_ref[g])
    xblk = jnp.where(m, xs_ref[...], 0.0)
    h = jnp.maximum(
        jnp.dot(xblk, w1_ref[0], preferred_element_type=jnp.float32), 0.0)
    y = jnp.dot(h, w2_ref[0], preferred_element_type=jnp.float32)

    first = (kd == 0) & ((g == 0) | (tile != tile_ref[jnp.maximum(g - 1, 0)]))

    @pl.when(first)
    def _():
        ys_ref[...] = y

    @pl.when(jnp.logical_not(first))
    def _():
        ys_ref[...] += y


def _gmm(xs, w1, w2, tile_of, e_of, lo, hi):
    spec = pltpu.PrefetchScalarGridSpec(
        num_scalar_prefetch=4,
        grid=(_G, _KD),
        in_specs=[
            pl.BlockSpec((_TM, _D), lambda g, kd, t, e, lo, hi: (t[g], 0)),
            pl.BlockSpec((1, _D, _TKD), lambda g, kd, t, e, lo, hi: (e[g], 0, kd)),
            pl.BlockSpec((1, _TKD, _D), lambda g, kd, t, e, lo, hi: (e[g], kd, 0)),
        ],
        out_specs=pl.BlockSpec((_TM, _D), lambda g, kd, t, e, lo, hi: (t[g], 0)),
    )
    return pl.pallas_call(
        _gmm_body,
        grid_spec=spec,
        out_shape=jax.ShapeDtypeStruct((_A, _D), jnp.float32),
    )(tile_of, e_of, lo, hi, xs, w1, w2)


def _combine_gather(ys, dest01):
    tpw = _N // _NW
    ch = 32
    nch = tpw // ch
    mesh = plsc.VectorSubcoreMesh(core_axis_name="c", subcore_axis_name="s")
    sds = jax.ShapeDtypeStruct((_N, _D), jnp.float32)

    @functools.partial(
        pl.kernel,
        out_type=(sds, sds),
        mesh=mesh,
        scratch_types=[
            pltpu.VMEM((ch, _D), jnp.float32),
            pltpu.VMEM((ch, _D), jnp.float32),
            pltpu.VMEM((ch,), jnp.int32),
            pltpu.VMEM((ch,), jnp.int32),
            pltpu.SemaphoreType.DMA,
            pltpu.SemaphoreType.DMA,
        ],
    )
    def k(ys_hbm, d_hbm, y0_hbm, y1_hbm, r0_v, r1_v, i0_v, i1_v, s0, s1):
        wid = lax.axis_index("s") * _NC + lax.axis_index("c")
        base = wid * tpw
        for cnk in range(nch):
            t0 = base + cnk * ch
            pltpu.sync_copy(d_hbm.at[0, pl.ds(t0, ch)], i0_v)
            pltpu.sync_copy(d_hbm.at[1, pl.ds(t0, ch)], i1_v)
            c0 = pltpu.async_copy(ys_hbm.at[i0_v], r0_v, s0)
            c1 = pltpu.async_copy(ys_hbm.at[i1_v], r1_v, s1)
            c0.wait()
            c1.wait()
            pltpu.sync_copy(r0_v, y0_hbm.at[pl.ds(t0, ch)])
            pltpu.sync_copy(r1_v, y1_hbm.at[pl.ds(t0, ch)])

    return k(ys, dest01)


def _combine_body(y0_ref, y1_ref, ew_ref, out_ref):
    ew = ew_ref[...]
    out_ref[...] = y0_ref[...] * ew[:, 0:1] + y1_ref[...] * ew[:, 1:2]


def _combine(y0, y1, ew):
    tme = 512
    return pl.pallas_call(
        _combine_body,
        grid=(_N // tme,),
        in_specs=[
            pl.BlockSpec((tme, _D), lambda i: (i, 0)),
            pl.BlockSpec((tme, _D), lambda i: (i, 0)),
            pl.BlockSpec((tme, _TOPK), lambda i: (i, 0)),
        ],
        out_specs=pl.BlockSpec((tme, _D), lambda i: (i, 0)),
        out_shape=jax.ShapeDtypeStruct((_N, _D), jnp.float32),
    )(y0, y1, ew)


def _route_meta(counts):
    counts = counts.astype(jnp.int32)
    offs = jnp.concatenate([jnp.zeros((1,), jnp.int32), jnp.cumsum(counts)])
    start_t = offs[:_E] // _TM
    end_t = jnp.where(counts > 0, (offs[1:] - 1) // _TM, start_t - 1)
    nt = jnp.maximum(end_t - start_t + 1, 0)
    unit_start = jnp.concatenate([jnp.zeros((1,), jnp.int32), jnp.cumsum(nt)])
    total_units = unit_start[_E]
    g = jnp.arange(_G, dtype=jnp.int32)
    e_id = jnp.sum(
        (g[:, None] >= unit_start[None, 1:]).astype(jnp.int32), axis=1)
    valid = g < total_units
    e_id = jnp.clip(e_id, 0, _E - 1)
    tile_real = start_t[e_id] + (g - unit_start[e_id])
    tile_of = jnp.where(valid, tile_real, _NT - 1)
    lo = jnp.where(valid, jnp.maximum(offs[e_id], tile_of * _TM), 0)
    hi = jnp.where(valid, jnp.minimum(offs[e_id + 1], (tile_of + 1) * _TM), 0)
    e_of = jnp.where(valid, e_id, 0)
    return tile_of, e_of, lo, hi


def kernel(x, expert_weights, W1, W2, expert_indices, batch_size_per_expert):
    ei2d = expert_indices.reshape(-1).astype(jnp.int32).reshape(128, 128)
    cnt = batch_size_per_expert.reshape(1, _E).astype(jnp.int32)

    dest2d = _dest(ei2d, cnt)
    dest01 = dest2d.reshape(_N, _TOPK).T

    xs = _dispatch(x, dest01)
    meta = _route_meta(batch_size_per_expert)
    ys = _gmm(xs, W1, W2, *meta)
    yu0, yu1 = _combine_gather(ys, dest01)
    return _combine(yu0, yu1, expert_weights)


# speedup vs baseline: 1.3412x; 1.0502x over previous
import functools

import jax
import jax.numpy as jnp
from jax import lax
from jax.experimental import pallas as pl
from jax.experimental.pallas import tpu as pltpu
import jax.experimental.pallas.tpu_sc as plsc

_E = 8
_TOPK = 2
_N = 8192
_D = 1024
_DFF = 4096
_A = _N * _TOPK

_TM = 1024
_TKD = 2048
_NT = _A // _TM
_G = _NT + _E - 1
_KD = _DFF // _TKD

_NC = 2
_NS = 16
_NW = _NC * _NS


def _dest_body(ei_ref, cnt_ref, dest_ref):
    ei = ei_ref[...]

    r = lax.broadcasted_iota(jnp.int32, (128, 128), 0)
    c = lax.broadcasted_iota(jnp.int32, (128, 128), 1)
    tlt = (r < c).astype(jnp.float32)
    slo = (c < r).astype(jnp.float32)

    dest = jnp.zeros((128, 128), jnp.float32)
    off = jnp.int32(0)
    for e in range(_E):
        plane = (ei == e).astype(jnp.float32)
        col_ex = jnp.dot(plane, tlt, preferred_element_type=jnp.float32)
        rowsum = jnp.sum(plane, axis=1, keepdims=True)
        row_ex = jnp.dot(slo, rowsum, preferred_element_type=jnp.float32)
        rank = col_ex + row_ex
        dest = dest + plane * (off.astype(jnp.float32) + rank)
        off = off + cnt_ref[0, e]
    dest_ref[...] = dest.astype(jnp.int32)


def _dest(ei2d, cnt):
    return pl.pallas_call(
        _dest_body,
        in_specs=[
            pl.BlockSpec((128, 128), lambda: (0, 0)),
            pl.BlockSpec(memory_space=pltpu.SMEM),
        ],
        out_shape=jax.ShapeDtypeStruct((128, 128), jnp.int32),
    )(ei2d, cnt)


def _dispatch(x, dest01):
    tpw = _N // _NW
    ch = 64
    nch = tpw // ch
    mesh = plsc.VectorSubcoreMesh(core_axis_name="c", subcore_axis_name="s")

    @functools.partial(
        pl.kernel,
        out_type=jax.ShapeDtypeStruct((_A, _D), jnp.float32),
        mesh=mesh,
        scratch_types=[
            pltpu.VMEM((ch, _D), jnp.float32),
            pltpu.VMEM((ch,), jnp.int32),
            pltpu.VMEM((ch,), jnp.int32),
            pltpu.SemaphoreType.DMA,
            pltpu.SemaphoreType.DMA,
        ],
    )
    def k(x_hbm, d_hbm, xs_hbm, rows_v, i0_v, i1_v, s0, s1):
        wid = lax.axis_index("s") * _NC + lax.axis_index("c")
        base = wid * tpw
        for cnk in range(nch):
            t0 = base + cnk * ch
            pltpu.sync_copy(x_hbm.at[pl.ds(t0, ch)], rows_v)
            pltpu.sync_copy(d_hbm.at[0, pl.ds(t0, ch)], i0_v)
            pltpu.sync_copy(d_hbm.at[1, pl.ds(t0, ch)], i1_v)
            c0 = pltpu.async_copy(rows_v, xs_hbm.at[i0_v], s0)
            c1 = pltpu.async_copy(rows_v, xs_hbm.at[i1_v], s1)
            c0.wait()
            c1.wait()

    return k(x, dest01)


def _gmm_body(tile_ref, e_ref, lo_ref, hi_ref, xs_ref, w1_ref, w2_ref, ys_ref):
    g = pl.program_id(0)
    kd = pl.program_id(1)
    tile = tile_ref[g]
    rid = tile * _TM + lax.broadcasted_iota(jnp.int32, (_TM, 1), 0)
    m = (rid >= lo_ref[g]) & (rid < hi_ref[g])
    xblk = jnp.where(m, xs_ref[...], 0.0)
    h = jnp.maximum(
        jnp.dot(xblk, w1_ref[0], preferred_element_type=jnp.float32), 0.0)
    y = jnp.dot(h, w2_ref[0], preferred_element_type=jnp.float32)

    first = (kd == 0) & ((g == 0) | (tile != tile_ref[jnp.maximum(g - 1, 0)]))

    @pl.when(first)
    def _():
        ys_ref[...] = y

    @pl.when(jnp.logical_not(first))
    def _():
        ys_ref[...] += y


def _gmm(xs, w1, w2, tile_of, e_of, lo, hi):
    spec = pltpu.PrefetchScalarGridSpec(
        num_scalar_prefetch=4,
        grid=(_G, _KD),
        in_specs=[
            pl.BlockSpec((_TM, _D), lambda g, kd, t, e, lo, hi: (t[g], 0)),
            pl.BlockSpec((1, _D, _TKD), lambda g, kd, t, e, lo, hi: (e[g], 0, kd)),
            pl.BlockSpec((1, _TKD, _D), lambda g, kd, t, e, lo, hi: (e[g], kd, 0)),
        ],
        out_specs=pl.BlockSpec((_TM, _D), lambda g, kd, t, e, lo, hi: (t[g], 0)),
    )
    return pl.pallas_call(
        _gmm_body,
        grid_spec=spec,
        out_shape=jax.ShapeDtypeStruct((_A, _D), jnp.float32),
    )(tile_of, e_of, lo, hi, xs, w1, w2)


def _combine_gather(ys, dest01):
    tpw = _N // _NW
    ch = 32
    nch = tpw // ch
    mesh = plsc.VectorSubcoreMesh(core_axis_name="c", subcore_axis_name="s")
    sds = jax.ShapeDtypeStruct((_N, _D), jnp.float32)

    @functools.partial(
        pl.kernel,
        out_type=(sds, sds),
        mesh=mesh,
        scratch_types=[
            pltpu.VMEM((ch, _D), jnp.float32),
            pltpu.VMEM((ch, _D), jnp.float32),
            pltpu.VMEM((ch,), jnp.int32),
            pltpu.VMEM((ch,), jnp.int32),
            pltpu.SemaphoreType.DMA,
            pltpu.SemaphoreType.DMA,
        ],
    )
    def k(ys_hbm, d_hbm, y0_hbm, y1_hbm, r0_v, r1_v, i0_v, i1_v, s0, s1):
        wid = lax.axis_index("s") * _NC + lax.axis_index("c")
        base = wid * tpw
        for cnk in range(nch):
            t0 = base + cnk * ch
            pltpu.sync_copy(d_hbm.at[0, pl.ds(t0, ch)], i0_v)
            pltpu.sync_copy(d_hbm.at[1, pl.ds(t0, ch)], i1_v)
            c0 = pltpu.async_copy(ys_hbm.at[i0_v], r0_v, s0)
            c1 = pltpu.async_copy(ys_hbm.at[i1_v], r1_v, s1)
            c0.wait()
            c1.wait()
            pltpu.sync_copy(r0_v, y0_hbm.at[pl.ds(t0, ch)])
            pltpu.sync_copy(r1_v, y1_hbm.at[pl.ds(t0, ch)])

    return k(ys, dest01)


def _combine_body(y0_ref, y1_ref, ew_ref, out_ref):
    ew = ew_ref[...]
    out_ref[...] = y0_ref[...] * ew[:, 0:1] + y1_ref[...] * ew[:, 1:2]


def _combine(y0, y1, ew):
    tme = 512
    return pl.pallas_call(
        _combine_body,
        grid=(_N // tme,),
        in_specs=[
            pl.BlockSpec((tme, _D), lambda i: (i, 0)),
            pl.BlockSpec((tme, _D), lambda i: (i, 0)),
            pl.BlockSpec((tme, _TOPK), lambda i: (i, 0)),
        ],
        out_specs=pl.BlockSpec((tme, _D), lambda i: (i, 0)),
        out_shape=jax.ShapeDtypeStruct((_N, _D), jnp.float32),
    )(y0, y1, ew)


def _route_meta(counts):
    counts = counts.astype(jnp.int32)
    offs = jnp.concatenate([jnp.zeros((1,), jnp.int32), jnp.cumsum(counts)])
    start_t = offs[:_E] // _TM
    end_t = jnp.where(counts > 0, (offs[1:] - 1) // _TM, start_t - 1)
    nt = jnp.maximum(end_t - start_t + 1, 0)
    unit_start = jnp.concatenate([jnp.zeros((1,), jnp.int32), jnp.cumsum(nt)])
    total_units = unit_start[_E]
    g = jnp.arange(_G, dtype=jnp.int32)
    e_id = jnp.sum(
        (g[:, None] >= unit_start[None, 1:]).astype(jnp.int32), axis=1)
    valid = g < total_units
    e_id = jnp.clip(e_id, 0, _E - 1)
    tile_real = start_t[e_id] + (g - unit_start[e_id])
    tile_of = jnp.where(valid, tile_real, _NT - 1)
    lo = jnp.where(valid, jnp.maximum(offs[e_id], tile_of * _TM), 0)
    hi = jnp.where(valid, jnp.minimum(offs[e_id + 1], (tile_of + 1) * _TM), 0)
    e_of = jnp.where(valid, e_id, 0)
    return tile_of, e_of, lo, hi


def kernel(x, expert_weights, W1, W2, expert_indices, batch_size_per_expert):
    ei2d = expert_indices.reshape(-1).astype(jnp.int32).reshape(128, 128)
    cnt = batch_size_per_expert.reshape(1, _E).astype(jnp.int32)

    dest2d = _dest(ei2d, cnt)
    dest01 = dest2d.reshape(_N, _TOPK).T

    xs = _dispatch(x, dest01)
    meta = _route_meta(batch_size_per_expert)
    ys = _gmm(xs, W1, W2, *meta)
    yu0, yu1 = _combine_gather(ys, dest01)
    return _combine(yu0, yu1, expert_weights)
